# Initial kernel scaffold; baseline (speedup 1.0000x reference)
import jax
import jax.numpy as jnp
from jax.experimental import pallas as pl


def kernel(objs, triples, boxes_gt, angles_gt, attributes, params):
    raise NotImplementedError("write your pallas kernel here")


# speedup vs baseline: 2.3580x; 2.3580x over previous
import functools

import jax
import jax.numpy as jnp
from jax import lax
from jax.experimental import pallas as pl
from jax.experimental.pallas import tpu as pltpu
from jax.experimental.pallas import tpu_sc as plsc

O = 10000
T = 160000
NUM_OBJS = 40
NUM_PREDS = 16
NUM_ATTRS = 10
NANGLE = 24
H = 512
DIN = 256

NC = 2
NS = 16
NW = NC * NS
KE = 128
NCH_E = T // KE
CW = 128
NCHUNK = H // CW
NPASS = NCHUNK // NC
OP = 10240
RPT = OP // NS
ZR = 32

EB = 1000
BN = 1000

_SC_MESH = plsc.VectorSubcoreMesh(
    core_axis_name="c", subcore_axis_name="s", num_cores=NC, num_subcores=NS)

_f32 = jnp.float32


def _dot(a, b):
    return jnp.dot(a, b, preferred_element_type=_f32)


def _dot_hi(a, b):
    return jnp.dot(a, b, preferred_element_type=_f32,
                   precision=lax.Precision.HIGHEST)


@functools.partial(
    pl.kernel,
    out_type=[jax.ShapeDtypeStruct((T, H), _f32),
              jax.ShapeDtypeStruct((T, H), _f32)],
    mesh=_SC_MESH,
    scratch_types=[pltpu.VMEM((KE,), jnp.int32),
                   pltpu.VMEM((KE, H), _f32),
                   pltpu.SemaphoreType.DMA],
)
def _sc_gather(us_hbm, uo_hbm, s_hbm, o_hbm, gs_hbm, go_hbm, idx_v, rows_v, sem):
    wid = lax.axis_index("s") * NC + lax.axis_index("c")
    ntrip = (NCH_E // NW) + jnp.where(wid < NCH_E % NW, 1, 0)

    def step(i, c):
        off = pl.multiple_of((wid + i * NW) * KE, KE)
        pltpu.sync_copy(s_hbm.at[pl.ds(off, KE)], idx_v)
        pltpu.async_copy(us_hbm.at[idx_v], rows_v, sem).wait()
        pltpu.sync_copy(rows_v, gs_hbm.at[pl.ds(off, KE)])
        pltpu.sync_copy(o_hbm.at[pl.ds(off, KE)], idx_v)
        pltpu.async_copy(uo_hbm.at[idx_v], rows_v, sem).wait()
        pltpu.sync_copy(rows_v, go_hbm.at[pl.ds(off, KE)])
        return c

    lax.fori_loop(0, ntrip, step, 0)


@functools.partial(
    pl.kernel,
    out_type=jax.ShapeDtypeStruct((NCHUNK, OP, CW), _f32),
    mesh=_SC_MESH,
    scratch_types=[pltpu.VMEM((KE,), jnp.int32),
                   pltpu.VMEM((KE, CW), _f32),
                   pltpu.VMEM((ZR, CW), _f32),
                   pltpu.VMEM_SHARED((OP, CW), _f32)],
)
def _sc_scatter(ns_hbm, no_hbm, s_hbm, o_hbm, pooled_hbm, idx_v, rows_v, zbuf, acc):
    cid = lax.axis_index("c")
    tid = lax.axis_index("s")
    zero = jnp.zeros((16,), _f32)
    for r in range(ZR):
        for cc in range(CW // 16):
            zbuf[r, cc * 16:(cc + 1) * 16] = zero
    row0 = tid * RPT
    ntrip = (NCH_E // NS) + jnp.where(tid < NCH_E % NS, 1, 0)
    for pp in range(NPASS):
        chunk = cid * NPASS + pp

        def zc(i, c):
            pltpu.sync_copy(zbuf, acc.at[pl.ds(pl.multiple_of(row0 + i * ZR, ZR), ZR)])
            return c

        lax.fori_loop(0, RPT // ZR, zc, 0)
        plsc.subcore_barrier()

        def sstep(i, c):
            off = pl.multiple_of((tid + i * NS) * KE, KE)
            pltpu.sync_copy(s_hbm.at[pl.ds(off, KE)], idx_v)
            pltpu.sync_copy(ns_hbm.at[chunk, pl.ds(off, KE)], rows_v)
            pltpu.sync_copy(rows_v, acc.at[idx_v], add=True)
            pltpu.sync_copy(o_hbm.at[pl.ds(off, KE)], idx_v)
            pltpu.sync_copy(no_hbm.at[chunk, pl.ds(off, KE)], rows_v)
            pltpu.sync_copy(rows_v, acc.at[idx_v], add=True)
            return c

        lax.fori_loop(0, ntrip, sstep, 0)
        plsc.subcore_barrier()

        def wb(i, c):
            r = pl.multiple_of(row0 + i * ZR, ZR)
            pltpu.sync_copy(acc.at[pl.ds(r, ZR)],
                            pooled_hbm.at[chunk, pl.ds(r, ZR)])
            return c

        lax.fori_loop(0, RPT // ZR, wb, 0)


@functools.partial(
    pl.kernel,
    out_type=jax.ShapeDtypeStruct((NC, OP, CW), _f32),
    mesh=_SC_MESH,
    scratch_types=[pltpu.VMEM((KE,), jnp.int32),
                   pltpu.VMEM((KE, CW), _f32),
                   pltpu.VMEM((ZR, CW), _f32),
                   pltpu.VMEM_SHARED((OP, CW), _f32)],
)
def _sc_counts(so_hbm, cnt_hbm, idx_v, ones_v, zbuf, acc):
    cid = lax.axis_index("c")
    tid = lax.axis_index("s")
    one = jnp.ones((16,), _f32)
    zero = jnp.zeros((16,), _f32)
    for r in range(KE):
        for cc in range(CW // 16):
            ones_v[r, cc * 16:(cc + 1) * 16] = one
    for r in range(ZR):
        for cc in range(CW // 16):
            zbuf[r, cc * 16:(cc + 1) * 16] = zero
    row0 = tid * RPT
    ntrip = (NCH_E // NS) + jnp.where(tid < NCH_E % NS, 1, 0)

    def zc(i, c):
        pltpu.sync_copy(zbuf, acc.at[pl.ds(pl.multiple_of(row0 + i * ZR, ZR), ZR)])
        return c

    lax.fori_loop(0, RPT // ZR, zc, 0)
    plsc.subcore_barrier()

    def sstep(i, c):
        off = pl.multiple_of(cid * T + (tid + i * NS) * KE, KE)
        pltpu.sync_copy(so_hbm.at[pl.ds(off, KE)], idx_v)
        pltpu.sync_copy(ones_v, acc.at[idx_v], add=True)
        return c

    lax.fori_loop(0, ntrip, sstep, 0)
    plsc.subcore_barrier()

    def wb(i, c):
        r = pl.multiple_of(row0 + i * ZR, ZR)
        pltpu.sync_copy(acc.at[pl.ds(r, ZR)], cnt_hbm.at[cid, pl.ds(r, ZR)])
        return c

    lax.fori_loop(0, RPT // ZR, wb, 0)


def _full_spec(shape):
    n = len(shape)
    return pl.BlockSpec(shape, lambda i, _n=n: (0,) * _n)


def _onehot(idx, n):
    return (idx == lax.broadcasted_iota(jnp.int32, (1, n), 1)).astype(_f32)


def _emb_select(idx_col, table_ref, n):
    acc = jnp.zeros((idx_col.shape[0], table_ref.shape[1]), _f32)
    for v in range(n):
        m = (idx_col == v).astype(_f32)
        acc = acc + m * table_ref[v, :][None, :]
    return acc


def _prologue(objs2, attrs2, angs2, boxes, obj_emb, attr_emb, angle_emb,
              box_w, box_b2):
    def body(objs_r, attrs_r, angs_r, boxes_r, oe_r, ae_r, ge_r, bw_r, bb_r,
             out_r):
        e_obj = _emb_select(objs_r[:], oe_r, NUM_OBJS + 1)
        e_attr = _emb_select(attrs_r[:], ae_r, NUM_ATTRS)
        e_ang = _emb_select(angs_r[:], ge_r, NANGLE)
        bv = _dot(boxes_r[:], bw_r[:]) + bb_r[:]
        out_r[:] = jnp.concatenate([e_obj, e_attr, bv, e_ang], axis=1)

    return pl.pallas_call(
        body,
        grid=(O // BN,),
        in_specs=[pl.BlockSpec((BN, 1), lambda i: (i, 0)),
                  pl.BlockSpec((BN, 1), lambda i: (i, 0)),
                  pl.BlockSpec((BN, 1), lambda i: (i, 0)),
                  pl.BlockSpec((BN, 6), lambda i: (i, 0)),
                  _full_spec(obj_emb.shape), _full_spec(attr_emb.shape),
                  _full_spec(angle_emb.shape), _full_spec(box_w.shape),
                  _full_spec(box_b2.shape)],
        out_specs=pl.BlockSpec((BN, DIN), lambda i: (i, 0)),
        out_shape=jax.ShapeDtypeStruct((O, DIN), _f32),
    )(objs2, attrs2, angs2, boxes, obj_emb, attr_emb, angle_emb, box_w, box_b2)


def _u_kernel(obj_vecs, w0s, w0o):
    def body(ov_r, ws_r, wo_r, us_r, uo_r):
        x = ov_r[:]
        us_r[:] = _dot(x, ws_r[:])
        uo_r[:] = _dot(x, wo_r[:])

    return pl.pallas_call(
        body,
        grid=(O // BN,),
        in_specs=[pl.BlockSpec((BN, DIN), lambda i: (i, 0)),
                  _full_spec(w0s.shape), _full_spec(w0o.shape)],
        out_specs=[pl.BlockSpec((BN, H), lambda i: (i, 0)),
                   pl.BlockSpec((BN, H), lambda i: (i, 0))],
        out_shape=[jax.ShapeDtypeStruct((O, H), _f32),
                   jax.ShapeDtypeStruct((O, H), _f32)],
    )(obj_vecs, w0s, w0o)


_EDGE_PARAMS = pltpu.CompilerParams(vmem_limit_bytes=100 * 1024 * 1024)


def _edge_mid(gs, go, pred, w0m, b02, w1, b12):
    def body(gs_r, go_r, pr_r, w0m_r, b0_r, w1_r, b1_r, nsc_r, noc_r, np_r):
        P = _dot(pr_r[:], w0m_r[:])
        h = jax.nn.relu(gs_r[:] + go_r[:] + P + b0_r[:])
        res = jax.nn.relu(_dot(h, w1_r[:]) + b1_r[:])
        for j in range(NCHUNK):
            nsc_r[j] = res[:, j * CW:(j + 1) * CW]
            noc_r[j] = res[:, H + DIN + j * CW:H + DIN + (j + 1) * CW]
        np_r[:] = res[:, H:H + DIN]

    return pl.pallas_call(
        body,
        grid=(T // EB,),
        in_specs=[pl.BlockSpec((EB, H), lambda i: (i, 0)),
                  pl.BlockSpec((EB, H), lambda i: (i, 0)),
                  pl.BlockSpec((EB, DIN), lambda i: (i, 0)),
                  _full_spec(w0m.shape), _full_spec(b02.shape),
                  _full_spec(w1.shape), _full_spec(b12.shape)],
        out_specs=[pl.BlockSpec((NCHUNK, EB, CW), lambda i: (0, i, 0)),
                   pl.BlockSpec((NCHUNK, EB, CW), lambda i: (0, i, 0)),
                   pl.BlockSpec((EB, DIN), lambda i: (i, 0))],
        out_shape=[jax.ShapeDtypeStruct((NCHUNK, T, CW), _f32),
                   jax.ShapeDtypeStruct((NCHUNK, T, CW), _f32),
                   jax.ShapeDtypeStruct((T, DIN), _f32)],
        compiler_params=_EDGE_PARAMS,
    )(gs, go, pred, w0m, b02, w1, b12)


def _edge_first(gs, go, p2, pred_emb, w0m, b02, w1, b12):
    def body(gs_r, go_r, p_r, pe_r, w0m_r, b0_r, w1_r, b1_r,
             nsc_r, noc_r, np_r):
        w16 = _dot(pe_r[:], w0m_r[:])
        P = _emb_select(p_r[:], w16, NUM_PREDS)
        h = jax.nn.relu(gs_r[:] + go_r[:] + P + b0_r[:])
        res = jax.nn.relu(_dot(h, w1_r[:]) + b1_r[:])
        for j in range(NCHUNK):
            nsc_r[j] = res[:, j * CW:(j + 1) * CW]
            noc_r[j] = res[:, H + DIN + j * CW:H + DIN + (j + 1) * CW]
        np_r[:] = res[:, H:H + DIN]

    return pl.pallas_call(
        body,
        grid=(T // EB,),
        in_specs=[pl.BlockSpec((EB, H), lambda i: (i, 0)),
                  pl.BlockSpec((EB, H), lambda i: (i, 0)),
                  pl.BlockSpec((EB, 1), lambda i: (i, 0)),
                  _full_spec(pred_emb.shape),
                  _full_spec(w0m.shape), _full_spec(b02.shape),
                  _full_spec(w1.shape), _full_spec(b12.shape)],
        out_specs=[pl.BlockSpec((NCHUNK, EB, CW), lambda i: (0, i, 0)),
                   pl.BlockSpec((NCHUNK, EB, CW), lambda i: (0, i, 0)),
                   pl.BlockSpec((EB, DIN), lambda i: (i, 0))],
        out_shape=[jax.ShapeDtypeStruct((NCHUNK, T, CW), _f32),
                   jax.ShapeDtypeStruct((NCHUNK, T, CW), _f32),
                   jax.ShapeDtypeStruct((T, DIN), _f32)],
        compiler_params=_EDGE_PARAMS,
    )(gs, go, p2, pred_emb, w0m, b02, w1, b12)


def _edge_last(gs, go, pred, w0m, b02, w1so, b1so2):
    def body(gs_r, go_r, pr_r, w0m_r, b0_r, w1_r, b1_r, nsc_r, noc_r):
        P = _dot(pr_r[:], w0m_r[:])
        h = jax.nn.relu(gs_r[:] + go_r[:] + P + b0_r[:])
        res = jax.nn.relu(_dot(h, w1_r[:]) + b1_r[:])
        for j in range(NCHUNK):
            nsc_r[j] = res[:, j * CW:(j + 1) * CW]
            noc_r[j] = res[:, H + j * CW:H + (j + 1) * CW]

    return pl.pallas_call(
        body,
        grid=(T // EB,),
        in_specs=[pl.BlockSpec((EB, H), lambda i: (i, 0)),
                  pl.BlockSpec((EB, H), lambda i: (i, 0)),
                  pl.BlockSpec((EB, DIN), lambda i: (i, 0)),
                  _full_spec(w0m.shape), _full_spec(b02.shape),
                  _full_spec(w1so.shape), _full_spec(b1so2.shape)],
        out_specs=[pl.BlockSpec((NCHUNK, EB, CW), lambda i: (0, i, 0)),
                   pl.BlockSpec((NCHUNK, EB, CW), lambda i: (0, i, 0))],
        out_shape=[jax.ShapeDtypeStruct((NCHUNK, T, CW), _f32),
                   jax.ShapeDtypeStruct((NCHUNK, T, CW), _f32)],
        compiler_params=_EDGE_PARAMS,
    )(gs, go, pred, w0m, b02, w1so, b1so2)


def _node(pooled, counts, w0, b02, w1, b12):
    def body(p_r, cnt_r, w0_r, b0_r, w1_r, b1_r, out_r):
        pool = jnp.concatenate([p_r[j] for j in range(NCHUNK)], axis=1)
        den = jnp.maximum(cnt_r[0, :, 0:1] + cnt_r[1, :, 0:1], 1.0)
        pool = pool / den
        h2 = jax.nn.relu(_dot(pool, w0_r[:]) + b0_r[:])
        out_r[:] = jax.nn.relu(_dot(h2, w1_r[:]) + b1_r[:])

    return pl.pallas_call(
        body,
        grid=(O // BN,),
        in_specs=[pl.BlockSpec((NCHUNK, BN, CW), lambda i: (0, i, 0)),
                  pl.BlockSpec((NC, BN, CW), lambda i: (0, i, 0)),
                  _full_spec(w0.shape), _full_spec(b02.shape),
                  _full_spec(w1.shape), _full_spec(b12.shape)],
        out_specs=pl.BlockSpec((BN, DIN), lambda i: (i, 0)),
        out_shape=jax.ShapeDtypeStruct((O, DIN), _f32),
    )(pooled, counts, w0, b02, w1, b12)


def _heads(obj_vecs, ws):
    def body(ov_r, bw0, bb0, bw1, bb1, bmw, bmb, bvw, bvb,
             aw0, ab0, aw1, ab1, amw, amb, avw, avb, mu_r, lv_r):
        x = ov_r[:]
        hb = jax.nn.relu(_dot(x, bw0[:]) + bb0[:])
        ob = jax.nn.relu(_dot(hb, bw1[:]) + bb1[:])
        ha = jax.nn.relu(_dot(x, aw0[:]) + ab0[:])
        oa = jax.nn.relu(_dot(ha, aw1[:]) + ab1[:])
        mu_r[:] = jnp.concatenate(
            [_dot(ob, bmw[:]) + bmb[:], _dot(oa, amw[:]) + amb[:]], axis=1)
        lv_r[:] = jnp.concatenate(
            [_dot(ob, bvw[:]) + bvb[:], _dot(oa, avw[:]) + avb[:]], axis=1)

    return pl.pallas_call(
        body,
        grid=(O // BN,),
        in_specs=[pl.BlockSpec((BN, DIN), lambda i: (i, 0))] +
                 [_full_spec(w.shape) for w in ws],
        out_specs=[pl.BlockSpec((BN, 128), lambda i: (i, 0)),
                   pl.BlockSpec((BN, 128), lambda i: (i, 0))],
        out_shape=[jax.ShapeDtypeStruct((O, 128), _f32),
                   jax.ShapeDtypeStruct((O, 128), _f32)],
    )(obj_vecs, *ws)


def kernel(objs, triples, boxes_gt, angles_gt, attributes, params):
    s = triples[:, 0].astype(jnp.int32)
    p = triples[:, 1].astype(jnp.int32)
    o = triples[:, 2].astype(jnp.int32)
    so = jnp.concatenate([s, o])

    counts = _sc_counts(so)

    obj_vecs = _prologue(
        objs.astype(jnp.int32)[:, None], attributes.astype(jnp.int32)[:, None],
        angles_gt.astype(jnp.int32)[:, None], boxes_gt,
        params['obj_emb'], params['attr_emb'], params['angle_emb'],
        params['box_w'], params['box_b'][None, :])

    pred_state = None
    for i in range(5):
        w0 = params['g%d_n1w0' % i]
        b02 = params['g%d_n1b0' % i][None, :]
        w1 = params['g%d_n1w1' % i]
        b1 = params['g%d_n1b1' % i]
        us, uo = _u_kernel(obj_vecs, w0[0:DIN], w0[2 * DIN:3 * DIN])
        gs, go = _sc_gather(us, uo, s, o)
        w0m = w0[DIN:2 * DIN]
        if i == 0:
            nsc, noc, pred_state = _edge_first(
                gs, go, p[:, None], params['pred_emb'], w0m, b02, w1,
                b1[None, :])
        elif i < 4:
            nsc, noc, pred_state = _edge_mid(
                gs, go, pred_state, w0m, b02, w1, b1[None, :])
        else:
            w1so = jnp.concatenate([w1[:, :H], w1[:, H + DIN:]], axis=1)
            b1so2 = jnp.concatenate([b1[:H], b1[H + DIN:]])[None, :]
            nsc, noc = _edge_last(gs, go, pred_state, w0m, b02, w1so, b1so2)
        pooled = _sc_scatter(nsc, noc, s, o)
        obj_vecs = _node(pooled, counts, params['g%d_n2w0' % i],
                         params['g%d_n2b0' % i][None, :],
                         params['g%d_n2w1' % i],
                         params['g%d_n2b1' % i][None, :])

    ws = [params['bmv_w0'], params['bmv_b0'][None, :],
          params['bmv_w1'], params['bmv_b1'][None, :],
          params['bm_w'], params['bm_b'][None, :],
          params['bv_w'], params['bv_b'][None, :],
          params['amv_w0'], params['amv_b0'][None, :],
          params['amv_w1'], params['amv_b1'][None, :],
          params['am_w'], params['am_b'][None, :],
          params['av_w'], params['av_b'][None, :]]
    mu, logvar = _heads(obj_vecs, ws)
    return mu, logvar


# speedup vs baseline: 2.7751x; 1.1769x over previous
import functools

import jax
import jax.numpy as jnp
from jax import lax
from jax.experimental import pallas as pl
from jax.experimental.pallas import tpu as pltpu
from jax.experimental.pallas import tpu_sc as plsc

O = 10000
T = 160000
NUM_OBJS = 40
NUM_PREDS = 16
NUM_ATTRS = 10
NANGLE = 24
H = 512
DIN = 256

NC = 2
NS = 16
NW = NC * NS
KE = 128
NCH_E = T // KE
CW = 128
NCHUNK = H // CW
NPASS = NCHUNK // NC
OP = 10240
RPT = OP // NS
ZR = 32

EB = 1000
BN = 1000

_SC_MESH = plsc.VectorSubcoreMesh(
    core_axis_name="c", subcore_axis_name="s", num_cores=NC, num_subcores=NS)

_f32 = jnp.float32


def _dot(a, b):
    return jnp.dot(a, b, preferred_element_type=_f32)


def _dot_hi(a, b):
    return jnp.dot(a, b, preferred_element_type=_f32,
                   precision=lax.Precision.HIGHEST)


@functools.partial(
    pl.kernel,
    out_type=[jax.ShapeDtypeStruct((2, T, H // 2), _f32),
              jax.ShapeDtypeStruct((2, T, H // 2), _f32)],
    mesh=_SC_MESH,
    scratch_types=[pltpu.VMEM((KE,), jnp.int32),
                   pltpu.VMEM((KE,), jnp.int32),
                   pltpu.VMEM((KE, H // 2), _f32),
                   pltpu.VMEM((KE, H // 2), _f32),
                   pltpu.VMEM((KE, H // 2), _f32),
                   pltpu.SemaphoreType.DMA,
                   pltpu.SemaphoreType.DMA,
                   pltpu.SemaphoreType.DMA,
                   pltpu.SemaphoreType.DMA,
                   pltpu.SemaphoreType.DMA,
                   pltpu.SemaphoreType.DMA],
)
def _sc_gather(us_lo, us_hi, uo_lo, uo_hi, s_hbm, o_hbm, gs_hbm, go_hbm,
               idx_s, idx_o, buf_a, buf_b, buf_c,
               sem_a, sem_b, sem_c, sem_wa, sem_wb, sem_wc):
    wid = lax.axis_index("s") * NC + lax.axis_index("c")
    ntrip = (NCH_E // NW) + jnp.where(wid < NCH_E % NW, 1, 0)

    def step(i, c):
        off = pl.multiple_of((wid + i * NW) * KE, KE)
        pltpu.sync_copy(s_hbm.at[pl.ds(off, KE)], idx_s)
        g_a = pltpu.async_copy(us_lo.at[idx_s], buf_a, sem_a)
        g_b = pltpu.async_copy(us_hi.at[idx_s], buf_b, sem_b)
        pltpu.sync_copy(o_hbm.at[pl.ds(off, KE)], idx_o)
        g_a.wait()
        w_a = pltpu.async_copy(buf_a, gs_hbm.at[0, pl.ds(off, KE)], sem_wa)
        g_c = pltpu.async_copy(uo_lo.at[idx_o], buf_c, sem_c)
        g_b.wait()
        w_b = pltpu.async_copy(buf_b, gs_hbm.at[1, pl.ds(off, KE)], sem_wb)
        g_c.wait()
        w_c = pltpu.async_copy(buf_c, go_hbm.at[0, pl.ds(off, KE)], sem_wc)
        w_a.wait()
        g_a2 = pltpu.async_copy(uo_hi.at[idx_o], buf_a, sem_a)
        g_a2.wait()
        pltpu.sync_copy(buf_a, go_hbm.at[1, pl.ds(off, KE)])
        w_b.wait()
        w_c.wait()
        return c

    lax.fori_loop(0, ntrip, step, 0)


@functools.partial(
    pl.kernel,
    out_type=jax.ShapeDtypeStruct((NCHUNK, OP, CW), _f32),
    mesh=_SC_MESH,
    scratch_types=[pltpu.VMEM((KE,), jnp.int32),
                   pltpu.VMEM((KE,), jnp.int32),
                   pltpu.VMEM((KE, CW), _f32),
                   pltpu.VMEM((KE, CW), _f32),
                   pltpu.VMEM((ZR, CW), _f32),
                   pltpu.VMEM_SHARED((OP, CW), _f32),
                   pltpu.SemaphoreType.DMA,
                   pltpu.SemaphoreType.DMA],
)
def _sc_scatter(ns_hbm, no_hbm, s_hbm, o_hbm, pooled_hbm,
                idx_a, idx_b, rows_a, rows_b, zbuf, acc, sem_a, sem_b):
    cid = lax.axis_index("c")
    tid = lax.axis_index("s")
    zero = jnp.zeros((16,), _f32)
    for r in range(ZR):
        for cc in range(CW // 16):
            zbuf[r, cc * 16:(cc + 1) * 16] = zero
    row0 = pl.multiple_of(tid * RPT, RPT)
    ntrip = (NCH_E // NS) + jnp.where(tid < NCH_E % NS, 1, 0)
    for pp in range(NPASS):
        chunk = cid * NPASS + pp

        def zc(i, c):
            pltpu.sync_copy(zbuf, acc.at[pl.ds(pl.multiple_of(row0 + i * ZR, ZR), ZR)])
            return c

        lax.fori_loop(0, RPT // ZR, zc, 0)
        plsc.subcore_barrier()

        def sstep(i, c):
            off = pl.multiple_of((tid + i * NS) * KE, KE)
            pltpu.sync_copy(s_hbm.at[pl.ds(off, KE)], idx_a)
            l_a = pltpu.async_copy(ns_hbm.at[chunk, pl.ds(off, KE)], rows_a, sem_a)
            pltpu.sync_copy(o_hbm.at[pl.ds(off, KE)], idx_b)
            l_b = pltpu.async_copy(no_hbm.at[chunk, pl.ds(off, KE)], rows_b, sem_b)
            l_a.wait()
            pltpu.sync_copy(rows_a, acc.at[idx_a], add=True)
            l_b.wait()
            pltpu.sync_copy(rows_b, acc.at[idx_b], add=True)
            return c

        lax.fori_loop(0, ntrip, sstep, 0)
        plsc.subcore_barrier()
        pltpu.sync_copy(acc.at[pl.ds(row0, RPT)],
                        pooled_hbm.at[chunk, pl.ds(row0, RPT)])


@functools.partial(
    pl.kernel,
    out_type=jax.ShapeDtypeStruct((NC, OP, CW), _f32),
    mesh=_SC_MESH,
    scratch_types=[pltpu.VMEM((KE,), jnp.int32),
                   pltpu.VMEM((KE, CW), _f32),
                   pltpu.VMEM((ZR, CW), _f32),
                   pltpu.VMEM_SHARED((OP, CW), _f32)],
)
def _sc_counts(so_hbm, cnt_hbm, idx_v, ones_v, zbuf, acc):
    cid = lax.axis_index("c")
    tid = lax.axis_index("s")
    one = jnp.ones((16,), _f32)
    zero = jnp.zeros((16,), _f32)
    for r in range(KE):
        for cc in range(CW // 16):
            ones_v[r, cc * 16:(cc + 1) * 16] = one
    for r in range(ZR):
        for cc in range(CW // 16):
            zbuf[r, cc * 16:(cc + 1) * 16] = zero
    row0 = tid * RPT
    ntrip = (NCH_E // NS) + jnp.where(tid < NCH_E % NS, 1, 0)

    def zc(i, c):
        pltpu.sync_copy(zbuf, acc.at[pl.ds(pl.multiple_of(row0 + i * ZR, ZR), ZR)])
        return c

    lax.fori_loop(0, RPT // ZR, zc, 0)
    plsc.subcore_barrier()

    def sstep(i, c):
        off = pl.multiple_of(cid * T + (tid + i * NS) * KE, KE)
        pltpu.sync_copy(so_hbm.at[pl.ds(off, KE)], idx_v)
        pltpu.sync_copy(ones_v, acc.at[idx_v], add=True)
        return c

    lax.fori_loop(0, ntrip, sstep, 0)
    plsc.subcore_barrier()

    def wb(i, c):
        r = pl.multiple_of(row0 + i * ZR, ZR)
        pltpu.sync_copy(acc.at[pl.ds(r, ZR)], cnt_hbm.at[cid, pl.ds(r, ZR)])
        return c

    lax.fori_loop(0, RPT // ZR, wb, 0)


def _full_spec(shape):
    n = len(shape)
    return pl.BlockSpec(shape, lambda i, _n=n: (0,) * _n)


def _onehot(idx, n):
    return (idx == lax.broadcasted_iota(jnp.int32, (1, n), 1)).astype(_f32)


def _emb_select(idx_col, table_ref, n):
    acc = jnp.zeros((idx_col.shape[0], table_ref.shape[1]), _f32)
    for v in range(n):
        m = (idx_col == v).astype(_f32)
        acc = acc + m * table_ref[v, :][None, :]
    return acc


def _prologue(objs2, attrs2, angs2, boxes, obj_emb, attr_emb, angle_emb,
              box_w, box_b2):
    def body(objs_r, attrs_r, angs_r, boxes_r, oe_r, ae_r, ge_r, bw_r, bb_r,
             out_r):
        e_obj = _emb_select(objs_r[:], oe_r, NUM_OBJS + 1)
        e_attr = _emb_select(attrs_r[:], ae_r, NUM_ATTRS)
        e_ang = _emb_select(angs_r[:], ge_r, NANGLE)
        bv = _dot(boxes_r[:], bw_r[:]) + bb_r[:]
        out_r[:] = jnp.concatenate([e_obj, e_attr, bv, e_ang], axis=1)

    return pl.pallas_call(
        body,
        grid=(O // BN,),
        in_specs=[pl.BlockSpec((BN, 1), lambda i: (i, 0)),
                  pl.BlockSpec((BN, 1), lambda i: (i, 0)),
                  pl.BlockSpec((BN, 1), lambda i: (i, 0)),
                  pl.BlockSpec((BN, 6), lambda i: (i, 0)),
                  _full_spec(obj_emb.shape), _full_spec(attr_emb.shape),
                  _full_spec(angle_emb.shape), _full_spec(box_w.shape),
                  _full_spec(box_b2.shape)],
        out_specs=pl.BlockSpec((BN, DIN), lambda i: (i, 0)),
        out_shape=jax.ShapeDtypeStruct((O, DIN), _f32),
    )(objs2, attrs2, angs2, boxes, obj_emb, attr_emb, angle_emb, box_w, box_b2)


def _u_kernel(obj_vecs, w0s, w0o):
    def body(ov_r, ws_r, wo_r, usl_r, ush_r, uol_r, uoh_r):
        x = ov_r[:]
        us = _dot(x, ws_r[:])
        uo = _dot(x, wo_r[:])
        usl_r[:] = us[:, :H // 2]
        ush_r[:] = us[:, H // 2:]
        uol_r[:] = uo[:, :H // 2]
        uoh_r[:] = uo[:, H // 2:]

    hspec = pl.BlockSpec((BN, H // 2), lambda i: (i, 0))
    hshape = jax.ShapeDtypeStruct((O, H // 2), _f32)
    return pl.pallas_call(
        body,
        grid=(O // BN,),
        in_specs=[pl.BlockSpec((BN, DIN), lambda i: (i, 0)),
                  _full_spec(w0s.shape), _full_spec(w0o.shape)],
        out_specs=[hspec, hspec, hspec, hspec],
        out_shape=[hshape, hshape, hshape, hshape],
    )(obj_vecs, w0s, w0o)


_EDGE_PARAMS = pltpu.CompilerParams(vmem_limit_bytes=100 * 1024 * 1024)


def _edge_mid(gs, go, pred, w0m, b02, w1, b12):
    def body(gs_r, go_r, pr_r, w0m_r, b0_r, w1_r, b1_r, nsc_r, noc_r, np_r):
        P = _dot(pr_r[:], w0m_r[:])
        g = (jnp.concatenate([gs_r[0], gs_r[1]], axis=1) +
             jnp.concatenate([go_r[0], go_r[1]], axis=1))
        h = jax.nn.relu(g + P + b0_r[:])
        res = jax.nn.relu(_dot(h, w1_r[:]) + b1_r[:])
        for j in range(NCHUNK):
            nsc_r[j] = res[:, j * CW:(j + 1) * CW]
            noc_r[j] = res[:, H + DIN + j * CW:H + DIN + (j + 1) * CW]
        np_r[:] = res[:, H:H + DIN]

    return pl.pallas_call(
        body,
        grid=(T // EB,),
        in_specs=[pl.BlockSpec((2, EB, H // 2), lambda i: (0, i, 0)),
                  pl.BlockSpec((2, EB, H // 2), lambda i: (0, i, 0)),
                  pl.BlockSpec((EB, DIN), lambda i: (i, 0)),
                  _full_spec(w0m.shape), _full_spec(b02.shape),
                  _full_spec(w1.shape), _full_spec(b12.shape)],
        out_specs=[pl.BlockSpec((NCHUNK, EB, CW), lambda i: (0, i, 0)),
                   pl.BlockSpec((NCHUNK, EB, CW), lambda i: (0, i, 0)),
                   pl.BlockSpec((EB, DIN), lambda i: (i, 0))],
        out_shape=[jax.ShapeDtypeStruct((NCHUNK, T, CW), _f32),
                   jax.ShapeDtypeStruct((NCHUNK, T, CW), _f32),
                   jax.ShapeDtypeStruct((T, DIN), _f32)],
        compiler_params=_EDGE_PARAMS,
    )(gs, go, pred, w0m, b02, w1, b12)


def _edge_first(gs, go, p2, pred_emb, w0m, b02, w1, b12):
    def body(gs_r, go_r, p_r, pe_r, w0m_r, b0_r, w1_r, b1_r,
             nsc_r, noc_r, np_r):
        w16 = _dot(pe_r[:], w0m_r[:])
        P = _emb_select(p_r[:], w16, NUM_PREDS)
        g = (jnp.concatenate([gs_r[0], gs_r[1]], axis=1) +
             jnp.concatenate([go_r[0], go_r[1]], axis=1))
        h = jax.nn.relu(g + P + b0_r[:])
        res = jax.nn.relu(_dot(h, w1_r[:]) + b1_r[:])
        for j in range(NCHUNK):
            nsc_r[j] = res[:, j * CW:(j + 1) * CW]
            noc_r[j] = res[:, H + DIN + j * CW:H + DIN + (j + 1) * CW]
        np_r[:] = res[:, H:H + DIN]

    return pl.pallas_call(
        body,
        grid=(T // EB,),
        in_specs=[pl.BlockSpec((2, EB, H // 2), lambda i: (0, i, 0)),
                  pl.BlockSpec((2, EB, H // 2), lambda i: (0, i, 0)),
                  pl.BlockSpec((EB, 1), lambda i: (i, 0)),
                  _full_spec(pred_emb.shape),
                  _full_spec(w0m.shape), _full_spec(b02.shape),
                  _full_spec(w1.shape), _full_spec(b12.shape)],
        out_specs=[pl.BlockSpec((NCHUNK, EB, CW), lambda i: (0, i, 0)),
                   pl.BlockSpec((NCHUNK, EB, CW), lambda i: (0, i, 0)),
                   pl.BlockSpec((EB, DIN), lambda i: (i, 0))],
        out_shape=[jax.ShapeDtypeStruct((NCHUNK, T, CW), _f32),
                   jax.ShapeDtypeStruct((NCHUNK, T, CW), _f32),
                   jax.ShapeDtypeStruct((T, DIN), _f32)],
        compiler_params=_EDGE_PARAMS,
    )(gs, go, p2, pred_emb, w0m, b02, w1, b12)


def _edge_last(gs, go, pred, w0m, b02, w1so, b1so2):
    def body(gs_r, go_r, pr_r, w0m_r, b0_r, w1_r, b1_r, nsc_r, noc_r):
        P = _dot(pr_r[:], w0m_r[:])
        g = (jnp.concatenate([gs_r[0], gs_r[1]], axis=1) +
             jnp.concatenate([go_r[0], go_r[1]], axis=1))
        h = jax.nn.relu(g + P + b0_r[:])
        res = jax.nn.relu(_dot(h, w1_r[:]) + b1_r[:])
        for j in range(NCHUNK):
            nsc_r[j] = res[:, j * CW:(j + 1) * CW]
            noc_r[j] = res[:, H + j * CW:H + (j + 1) * CW]

    return pl.pallas_call(
        body,
        grid=(T // EB,),
        in_specs=[pl.BlockSpec((2, EB, H // 2), lambda i: (0, i, 0)),
                  pl.BlockSpec((2, EB, H // 2), lambda i: (0, i, 0)),
                  pl.BlockSpec((EB, DIN), lambda i: (i, 0)),
                  _full_spec(w0m.shape), _full_spec(b02.shape),
                  _full_spec(w1so.shape), _full_spec(b1so2.shape)],
        out_specs=[pl.BlockSpec((NCHUNK, EB, CW), lambda i: (0, i, 0)),
                   pl.BlockSpec((NCHUNK, EB, CW), lambda i: (0, i, 0))],
        out_shape=[jax.ShapeDtypeStruct((NCHUNK, T, CW), _f32),
                   jax.ShapeDtypeStruct((NCHUNK, T, CW), _f32)],
        compiler_params=_EDGE_PARAMS,
    )(gs, go, pred, w0m, b02, w1so, b1so2)


def _node(pooled, counts, w0, b02, w1, b12):
    def body(p_r, cnt_r, w0_r, b0_r, w1_r, b1_r, out_r):
        pool = jnp.concatenate([p_r[j] for j in range(NCHUNK)], axis=1)
        den = jnp.maximum(cnt_r[0, :, 0:1] + cnt_r[1, :, 0:1], 1.0)
        pool = pool / den
        h2 = jax.nn.relu(_dot(pool, w0_r[:]) + b0_r[:])
        out_r[:] = jax.nn.relu(_dot(h2, w1_r[:]) + b1_r[:])

    return pl.pallas_call(
        body,
        grid=(O // BN,),
        in_specs=[pl.BlockSpec((NCHUNK, BN, CW), lambda i: (0, i, 0)),
                  pl.BlockSpec((NC, BN, CW), lambda i: (0, i, 0)),
                  _full_spec(w0.shape), _full_spec(b02.shape),
                  _full_spec(w1.shape), _full_spec(b12.shape)],
        out_specs=pl.BlockSpec((BN, DIN), lambda i: (i, 0)),
        out_shape=jax.ShapeDtypeStruct((O, DIN), _f32),
    )(pooled, counts, w0, b02, w1, b12)


def _heads(obj_vecs, ws):
    def body(ov_r, bw0, bb0, bw1, bb1, bmw, bmb, bvw, bvb,
             aw0, ab0, aw1, ab1, amw, amb, avw, avb, mu_r, lv_r):
        x = ov_r[:]
        hb = jax.nn.relu(_dot(x, bw0[:]) + bb0[:])
        ob = jax.nn.relu(_dot(hb, bw1[:]) + bb1[:])
        ha = jax.nn.relu(_dot(x, aw0[:]) + ab0[:])
        oa = jax.nn.relu(_dot(ha, aw1[:]) + ab1[:])
        mu_r[:] = jnp.concatenate(
            [_dot(ob, bmw[:]) + bmb[:], _dot(oa, amw[:]) + amb[:]], axis=1)
        lv_r[:] = jnp.concatenate(
            [_dot(ob, bvw[:]) + bvb[:], _dot(oa, avw[:]) + avb[:]], axis=1)

    return pl.pallas_call(
        body,
        grid=(O // BN,),
        in_specs=[pl.BlockSpec((BN, DIN), lambda i: (i, 0))] +
                 [_full_spec(w.shape) for w in ws],
        out_specs=[pl.BlockSpec((BN, 128), lambda i: (i, 0)),
                   pl.BlockSpec((BN, 128), lambda i: (i, 0))],
        out_shape=[jax.ShapeDtypeStruct((O, 128), _f32),
                   jax.ShapeDtypeStruct((O, 128), _f32)],
    )(obj_vecs, *ws)


def kernel(objs, triples, boxes_gt, angles_gt, attributes, params):
    s = triples[:, 0].astype(jnp.int32)
    p = triples[:, 1].astype(jnp.int32)
    o = triples[:, 2].astype(jnp.int32)
    so = jnp.concatenate([s, o])

    counts = _sc_counts(so)

    obj_vecs = _prologue(
        objs.astype(jnp.int32)[:, None], attributes.astype(jnp.int32)[:, None],
        angles_gt.astype(jnp.int32)[:, None], boxes_gt,
        params['obj_emb'], params['attr_emb'], params['angle_emb'],
        params['box_w'], params['box_b'][None, :])

    pred_state = None
    for i in range(5):
        w0 = params['g%d_n1w0' % i]
        b02 = params['g%d_n1b0' % i][None, :]
        w1 = params['g%d_n1w1' % i]
        b1 = params['g%d_n1b1' % i]
        usl, ush, uol, uoh = _u_kernel(obj_vecs, w0[0:DIN], w0[2 * DIN:3 * DIN])
        gs, go = _sc_gather(usl, ush, uol, uoh, s, o)
        w0m = w0[DIN:2 * DIN]
        if i == 0:
            nsc, noc, pred_state = _edge_first(
                gs, go, p[:, None], params['pred_emb'], w0m, b02, w1,
                b1[None, :])
        elif i < 4:
            nsc, noc, pred_state = _edge_mid(
                gs, go, pred_state, w0m, b02, w1, b1[None, :])
        else:
            w1so = jnp.concatenate([w1[:, :H], w1[:, H + DIN:]], axis=1)
            b1so2 = jnp.concatenate([b1[:H], b1[H + DIN:]])[None, :]
            nsc, noc = _edge_last(gs, go, pred_state, w0m, b02, w1so, b1so2)
        pooled = _sc_scatter(nsc, noc, s, o)
        obj_vecs = _node(pooled, counts, params['g%d_n2w0' % i],
                         params['g%d_n2b0' % i][None, :],
                         params['g%d_n2w1' % i],
                         params['g%d_n2b1' % i][None, :])

    ws = [params['bmv_w0'], params['bmv_b0'][None, :],
          params['bmv_w1'], params['bmv_b1'][None, :],
          params['bm_w'], params['bm_b'][None, :],
          params['bv_w'], params['bv_b'][None, :],
          params['amv_w0'], params['amv_b0'][None, :],
          params['amv_w1'], params['amv_b1'][None, :],
          params['am_w'], params['am_b'][None, :],
          params['av_w'], params['av_b'][None, :]]
    mu, logvar = _heads(obj_vecs, ws)
    return mu, logvar


# speedup vs baseline: 2.7842x; 1.0033x over previous
import functools

import jax
import jax.numpy as jnp
from jax import lax
from jax.experimental import pallas as pl
from jax.experimental.pallas import tpu as pltpu
from jax.experimental.pallas import tpu_sc as plsc

O = 10000
T = 160000
NUM_OBJS = 40
NUM_PREDS = 16
NUM_ATTRS = 10
NANGLE = 24
H = 512
DIN = 256

NC = 2
NS = 16
NW = NC * NS
KE = 128
NCH_E = T // KE
CW = 128
NCHUNK = H // CW
NPASS = NCHUNK // NC
OP = 10240
RPT = OP // NS
ZR = 32

EB = 1000
BN = 1000

_SC_MESH = plsc.VectorSubcoreMesh(
    core_axis_name="c", subcore_axis_name="s", num_cores=NC, num_subcores=NS)

_f32 = jnp.float32


def _dot(a, b):
    return jnp.dot(a, b, preferred_element_type=_f32)


def _dot_hi(a, b):
    return jnp.dot(a, b, preferred_element_type=_f32,
                   precision=lax.Precision.HIGHEST)


@functools.partial(
    pl.kernel,
    out_type=[jax.ShapeDtypeStruct((2, T, H // 2), _f32),
              jax.ShapeDtypeStruct((2, T, H // 2), _f32)],
    mesh=_SC_MESH,
    scratch_types=[pltpu.VMEM((KE,), jnp.int32),
                   pltpu.VMEM((KE,), jnp.int32),
                   pltpu.VMEM((KE, H // 2), _f32),
                   pltpu.VMEM((KE, H // 2), _f32),
                   pltpu.VMEM((KE, H // 2), _f32),
                   pltpu.SemaphoreType.DMA,
                   pltpu.SemaphoreType.DMA,
                   pltpu.SemaphoreType.DMA,
                   pltpu.SemaphoreType.DMA,
                   pltpu.SemaphoreType.DMA,
                   pltpu.SemaphoreType.DMA],
)
def _sc_gather(us_lo, us_hi, uo_lo, uo_hi, s_hbm, o_hbm, gs_hbm, go_hbm,
               idx_s, idx_o, buf_a, buf_b, buf_c,
               sem_a, sem_b, sem_c, sem_wa, sem_wb, sem_wc):
    wid = lax.axis_index("s") * NC + lax.axis_index("c")
    ntrip = (NCH_E // NW) + jnp.where(wid < NCH_E % NW, 1, 0)

    def step(i, c):
        off = pl.multiple_of((wid + i * NW) * KE, KE)
        pltpu.sync_copy(s_hbm.at[pl.ds(off, KE)], idx_s)
        g_a = pltpu.async_copy(us_lo.at[idx_s], buf_a, sem_a)
        g_b = pltpu.async_copy(us_hi.at[idx_s], buf_b, sem_b)
        pltpu.sync_copy(o_hbm.at[pl.ds(off, KE)], idx_o)
        g_a.wait()
        w_a = pltpu.async_copy(buf_a, gs_hbm.at[0, pl.ds(off, KE)], sem_wa)
        g_c = pltpu.async_copy(uo_lo.at[idx_o], buf_c, sem_c)
        g_b.wait()
        w_b = pltpu.async_copy(buf_b, gs_hbm.at[1, pl.ds(off, KE)], sem_wb)
        g_c.wait()
        w_c = pltpu.async_copy(buf_c, go_hbm.at[0, pl.ds(off, KE)], sem_wc)
        w_a.wait()
        g_a2 = pltpu.async_copy(uo_hi.at[idx_o], buf_a, sem_a)
        g_a2.wait()
        pltpu.sync_copy(buf_a, go_hbm.at[1, pl.ds(off, KE)])
        w_b.wait()
        w_c.wait()
        return c

    lax.fori_loop(0, ntrip, step, 0)


@functools.partial(
    pl.kernel,
    out_type=jax.ShapeDtypeStruct((NCHUNK, OP, CW), _f32),
    mesh=_SC_MESH,
    scratch_types=[pltpu.VMEM((KE,), jnp.int32),
                   pltpu.VMEM((KE,), jnp.int32),
                   pltpu.VMEM((KE, CW), _f32),
                   pltpu.VMEM((KE, CW), _f32),
                   pltpu.VMEM((ZR, CW), _f32),
                   pltpu.VMEM_SHARED((OP, CW), _f32),
                   pltpu.SemaphoreType.DMA,
                   pltpu.SemaphoreType.DMA,
                   pltpu.SemaphoreType.DMA,
                   pltpu.SemaphoreType.DMA],
)
def _sc_scatter(ns_hbm, no_hbm, s_hbm, o_hbm, pooled_hbm,
                idx_a, idx_b, rows_a, rows_b, zbuf, acc,
                sem_a, sem_b, sem_sa, sem_sb):
    cid = lax.axis_index("c")
    tid = lax.axis_index("s")
    zero = jnp.zeros((16,), _f32)
    for r in range(ZR):
        for cc in range(CW // 16):
            zbuf[r, cc * 16:(cc + 1) * 16] = zero
    row0 = pl.multiple_of(tid * RPT, RPT)
    ntrip = (NCH_E // NS) + jnp.where(tid < NCH_E % NS, 1, 0)
    for pp in range(NPASS):
        chunk = cid * NPASS + pp

        def zc(i, c):
            pltpu.sync_copy(zbuf, acc.at[pl.ds(pl.multiple_of(row0 + i * ZR, ZR), ZR)])
            return c

        lax.fori_loop(0, RPT // ZR, zc, 0)
        plsc.subcore_barrier()

        def sstep(i, c):
            off = pl.multiple_of((tid + i * NS) * KE, KE)
            pltpu.sync_copy(s_hbm.at[pl.ds(off, KE)], idx_a)
            l_a = pltpu.async_copy(ns_hbm.at[chunk, pl.ds(off, KE)], rows_a, sem_a)
            pltpu.sync_copy(o_hbm.at[pl.ds(off, KE)], idx_b)
            l_b = pltpu.async_copy(no_hbm.at[chunk, pl.ds(off, KE)], rows_b, sem_b)
            l_a.wait()
            s_a = pltpu.async_copy(rows_a, acc.at[idx_a], sem_sa, add=True)
            l_b.wait()
            s_b = pltpu.async_copy(rows_b, acc.at[idx_b], sem_sb, add=True)
            s_a.wait()
            s_b.wait()
            return c

        lax.fori_loop(0, ntrip, sstep, 0)
        plsc.subcore_barrier()
        pltpu.sync_copy(acc.at[pl.ds(row0, RPT)],
                        pooled_hbm.at[chunk, pl.ds(row0, RPT)])


@functools.partial(
    pl.kernel,
    out_type=jax.ShapeDtypeStruct((NC, OP, CW), _f32),
    mesh=_SC_MESH,
    scratch_types=[pltpu.VMEM((KE,), jnp.int32),
                   pltpu.VMEM((KE, CW), _f32),
                   pltpu.VMEM((ZR, CW), _f32),
                   pltpu.VMEM_SHARED((OP, CW), _f32)],
)
def _sc_counts(so_hbm, cnt_hbm, idx_v, ones_v, zbuf, acc):
    cid = lax.axis_index("c")
    tid = lax.axis_index("s")
    one = jnp.ones((16,), _f32)
    zero = jnp.zeros((16,), _f32)
    for r in range(KE):
        for cc in range(CW // 16):
            ones_v[r, cc * 16:(cc + 1) * 16] = one
    for r in range(ZR):
        for cc in range(CW // 16):
            zbuf[r, cc * 16:(cc + 1) * 16] = zero
    row0 = tid * RPT
    ntrip = (NCH_E // NS) + jnp.where(tid < NCH_E % NS, 1, 0)

    def zc(i, c):
        pltpu.sync_copy(zbuf, acc.at[pl.ds(pl.multiple_of(row0 + i * ZR, ZR), ZR)])
        return c

    lax.fori_loop(0, RPT // ZR, zc, 0)
    plsc.subcore_barrier()

    def sstep(i, c):
        off = pl.multiple_of(cid * T + (tid + i * NS) * KE, KE)
        pltpu.sync_copy(so_hbm.at[pl.ds(off, KE)], idx_v)
        pltpu.sync_copy(ones_v, acc.at[idx_v], add=True)
        return c

    lax.fori_loop(0, ntrip, sstep, 0)
    plsc.subcore_barrier()

    def wb(i, c):
        r = pl.multiple_of(row0 + i * ZR, ZR)
        pltpu.sync_copy(acc.at[pl.ds(r, ZR)], cnt_hbm.at[cid, pl.ds(r, ZR)])
        return c

    lax.fori_loop(0, RPT // ZR, wb, 0)


def _full_spec(shape):
    n = len(shape)
    return pl.BlockSpec(shape, lambda i, _n=n: (0,) * _n)


def _onehot(idx, n):
    return (idx == lax.broadcasted_iota(jnp.int32, (1, n), 1)).astype(_f32)


def _emb_select(idx_col, table_ref, n):
    acc = jnp.zeros((idx_col.shape[0], table_ref.shape[1]), _f32)
    for v in range(n):
        m = (idx_col == v).astype(_f32)
        acc = acc + m * table_ref[v, :][None, :]
    return acc


def _prologue(objs2, attrs2, angs2, boxes, obj_emb, attr_emb, angle_emb,
              box_w, box_b2):
    def body(objs_r, attrs_r, angs_r, boxes_r, oe_r, ae_r, ge_r, bw_r, bb_r,
             out_r):
        e_obj = _emb_select(objs_r[:], oe_r, NUM_OBJS + 1)
        e_attr = _emb_select(attrs_r[:], ae_r, NUM_ATTRS)
        e_ang = _emb_select(angs_r[:], ge_r, NANGLE)
        bv = _dot(boxes_r[:], bw_r[:]) + bb_r[:]
        out_r[:] = jnp.concatenate([e_obj, e_attr, bv, e_ang], axis=1)

    return pl.pallas_call(
        body,
        grid=(O // BN,),
        in_specs=[pl.BlockSpec((BN, 1), lambda i: (i, 0)),
                  pl.BlockSpec((BN, 1), lambda i: (i, 0)),
                  pl.BlockSpec((BN, 1), lambda i: (i, 0)),
                  pl.BlockSpec((BN, 6), lambda i: (i, 0)),
                  _full_spec(obj_emb.shape), _full_spec(attr_emb.shape),
                  _full_spec(angle_emb.shape), _full_spec(box_w.shape),
                  _full_spec(box_b2.shape)],
        out_specs=pl.BlockSpec((BN, DIN), lambda i: (i, 0)),
        out_shape=jax.ShapeDtypeStruct((O, DIN), _f32),
    )(objs2, attrs2, angs2, boxes, obj_emb, attr_emb, angle_emb, box_w, box_b2)


def _u_kernel(obj_vecs, w0s, w0o):
    def body(ov_r, ws_r, wo_r, usl_r, ush_r, uol_r, uoh_r):
        x = ov_r[:]
        us = _dot(x, ws_r[:])
        uo = _dot(x, wo_r[:])
        usl_r[:] = us[:, :H // 2]
        ush_r[:] = us[:, H // 2:]
        uol_r[:] = uo[:, :H // 2]
        uoh_r[:] = uo[:, H // 2:]

    hspec = pl.BlockSpec((BN, H // 2), lambda i: (i, 0))
    hshape = jax.ShapeDtypeStruct((O, H // 2), _f32)
    return pl.pallas_call(
        body,
        grid=(O // BN,),
        in_specs=[pl.BlockSpec((BN, DIN), lambda i: (i, 0)),
                  _full_spec(w0s.shape), _full_spec(w0o.shape)],
        out_specs=[hspec, hspec, hspec, hspec],
        out_shape=[hshape, hshape, hshape, hshape],
    )(obj_vecs, w0s, w0o)


_EDGE_PARAMS = pltpu.CompilerParams(vmem_limit_bytes=100 * 1024 * 1024)


def _edge_mid(gs, go, pred, w0m, b02, w1, b12):
    def body(gs_r, go_r, pr_r, w0m_r, b0_r, w1_r, b1_r, nsc_r, noc_r, np_r):
        P = _dot(pr_r[:], w0m_r[:])
        g = (jnp.concatenate([gs_r[0], gs_r[1]], axis=1) +
             jnp.concatenate([go_r[0], go_r[1]], axis=1))
        h = jax.nn.relu(g + P + b0_r[:])
        res = jax.nn.relu(_dot(h, w1_r[:]) + b1_r[:])
        for j in range(NCHUNK):
            nsc_r[j] = res[:, j * CW:(j + 1) * CW]
            noc_r[j] = res[:, H + DIN + j * CW:H + DIN + (j + 1) * CW]
        np_r[:] = res[:, H:H + DIN]

    return pl.pallas_call(
        body,
        grid=(T // EB,),
        in_specs=[pl.BlockSpec((2, EB, H // 2), lambda i: (0, i, 0)),
                  pl.BlockSpec((2, EB, H // 2), lambda i: (0, i, 0)),
                  pl.BlockSpec((EB, DIN), lambda i: (i, 0)),
                  _full_spec(w0m.shape), _full_spec(b02.shape),
                  _full_spec(w1.shape), _full_spec(b12.shape)],
        out_specs=[pl.BlockSpec((NCHUNK, EB, CW), lambda i: (0, i, 0)),
                   pl.BlockSpec((NCHUNK, EB, CW), lambda i: (0, i, 0)),
                   pl.BlockSpec((EB, DIN), lambda i: (i, 0))],
        out_shape=[jax.ShapeDtypeStruct((NCHUNK, T, CW), _f32),
                   jax.ShapeDtypeStruct((NCHUNK, T, CW), _f32),
                   jax.ShapeDtypeStruct((T, DIN), _f32)],
        compiler_params=_EDGE_PARAMS,
    )(gs, go, pred, w0m, b02, w1, b12)


def _edge_first(gs, go, p2, pred_emb, w0m, b02, w1, b12):
    def body(gs_r, go_r, p_r, pe_r, w0m_r, b0_r, w1_r, b1_r,
             nsc_r, noc_r, np_r):
        w16 = _dot(pe_r[:], w0m_r[:])
        P = _emb_select(p_r[:], w16, NUM_PREDS)
        g = (jnp.concatenate([gs_r[0], gs_r[1]], axis=1) +
             jnp.concatenate([go_r[0], go_r[1]], axis=1))
        h = jax.nn.relu(g + P + b0_r[:])
        res = jax.nn.relu(_dot(h, w1_r[:]) + b1_r[:])
        for j in range(NCHUNK):
            nsc_r[j] = res[:, j * CW:(j + 1) * CW]
            noc_r[j] = res[:, H + DIN + j * CW:H + DIN + (j + 1) * CW]
        np_r[:] = res[:, H:H + DIN]

    return pl.pallas_call(
        body,
        grid=(T // EB,),
        in_specs=[pl.BlockSpec((2, EB, H // 2), lambda i: (0, i, 0)),
                  pl.BlockSpec((2, EB, H // 2), lambda i: (0, i, 0)),
                  pl.BlockSpec((EB, 1), lambda i: (i, 0)),
                  _full_spec(pred_emb.shape),
                  _full_spec(w0m.shape), _full_spec(b02.shape),
                  _full_spec(w1.shape), _full_spec(b12.shape)],
        out_specs=[pl.BlockSpec((NCHUNK, EB, CW), lambda i: (0, i, 0)),
                   pl.BlockSpec((NCHUNK, EB, CW), lambda i: (0, i, 0)),
                   pl.BlockSpec((EB, DIN), lambda i: (i, 0))],
        out_shape=[jax.ShapeDtypeStruct((NCHUNK, T, CW), _f32),
                   jax.ShapeDtypeStruct((NCHUNK, T, CW), _f32),
                   jax.ShapeDtypeStruct((T, DIN), _f32)],
        compiler_params=_EDGE_PARAMS,
    )(gs, go, p2, pred_emb, w0m, b02, w1, b12)


def _edge_last(gs, go, pred, w0m, b02, w1so, b1so2):
    def body(gs_r, go_r, pr_r, w0m_r, b0_r, w1_r, b1_r, nsc_r, noc_r):
        P = _dot(pr_r[:], w0m_r[:])
        g = (jnp.concatenate([gs_r[0], gs_r[1]], axis=1) +
             jnp.concatenate([go_r[0], go_r[1]], axis=1))
        h = jax.nn.relu(g + P + b0_r[:])
        res = jax.nn.relu(_dot(h, w1_r[:]) + b1_r[:])
        for j in range(NCHUNK):
            nsc_r[j] = res[:, j * CW:(j + 1) * CW]
            noc_r[j] = res[:, H + j * CW:H + (j + 1) * CW]

    return pl.pallas_call(
        body,
        grid=(T // EB,),
        in_specs=[pl.BlockSpec((2, EB, H // 2), lambda i: (0, i, 0)),
                  pl.BlockSpec((2, EB, H // 2), lambda i: (0, i, 0)),
                  pl.BlockSpec((EB, DIN), lambda i: (i, 0)),
                  _full_spec(w0m.shape), _full_spec(b02.shape),
                  _full_spec(w1so.shape), _full_spec(b1so2.shape)],
        out_specs=[pl.BlockSpec((NCHUNK, EB, CW), lambda i: (0, i, 0)),
                   pl.BlockSpec((NCHUNK, EB, CW), lambda i: (0, i, 0))],
        out_shape=[jax.ShapeDtypeStruct((NCHUNK, T, CW), _f32),
                   jax.ShapeDtypeStruct((NCHUNK, T, CW), _f32)],
        compiler_params=_EDGE_PARAMS,
    )(gs, go, pred, w0m, b02, w1so, b1so2)


def _node(pooled, counts, w0, b02, w1, b12):
    def body(p_r, cnt_r, w0_r, b0_r, w1_r, b1_r, out_r):
        pool = jnp.concatenate([p_r[j] for j in range(NCHUNK)], axis=1)
        den = jnp.maximum(cnt_r[0, :, 0:1] + cnt_r[1, :, 0:1], 1.0)
        pool = pool / den
        h2 = jax.nn.relu(_dot(pool, w0_r[:]) + b0_r[:])
        out_r[:] = jax.nn.relu(_dot(h2, w1_r[:]) + b1_r[:])

    return pl.pallas_call(
        body,
        grid=(O // BN,),
        in_specs=[pl.BlockSpec((NCHUNK, BN, CW), lambda i: (0, i, 0)),
                  pl.BlockSpec((NC, BN, CW), lambda i: (0, i, 0)),
                  _full_spec(w0.shape), _full_spec(b02.shape),
                  _full_spec(w1.shape), _full_spec(b12.shape)],
        out_specs=pl.BlockSpec((BN, DIN), lambda i: (i, 0)),
        out_shape=jax.ShapeDtypeStruct((O, DIN), _f32),
    )(pooled, counts, w0, b02, w1, b12)


def _heads(obj_vecs, ws):
    def body(ov_r, bw0, bb0, bw1, bb1, bmw, bmb, bvw, bvb,
             aw0, ab0, aw1, ab1, amw, amb, avw, avb, mu_r, lv_r):
        x = ov_r[:]
        hb = jax.nn.relu(_dot(x, bw0[:]) + bb0[:])
        ob = jax.nn.relu(_dot(hb, bw1[:]) + bb1[:])
        ha = jax.nn.relu(_dot(x, aw0[:]) + ab0[:])
        oa = jax.nn.relu(_dot(ha, aw1[:]) + ab1[:])
        mu_r[:] = jnp.concatenate(
            [_dot(ob, bmw[:]) + bmb[:], _dot(oa, amw[:]) + amb[:]], axis=1)
        lv_r[:] = jnp.concatenate(
            [_dot(ob, bvw[:]) + bvb[:], _dot(oa, avw[:]) + avb[:]], axis=1)

    return pl.pallas_call(
        body,
        grid=(O // BN,),
        in_specs=[pl.BlockSpec((BN, DIN), lambda i: (i, 0))] +
                 [_full_spec(w.shape) for w in ws],
        out_specs=[pl.BlockSpec((BN, 128), lambda i: (i, 0)),
                   pl.BlockSpec((BN, 128), lambda i: (i, 0))],
        out_shape=[jax.ShapeDtypeStruct((O, 128), _f32),
                   jax.ShapeDtypeStruct((O, 128), _f32)],
    )(obj_vecs, *ws)


def kernel(objs, triples, boxes_gt, angles_gt, attributes, params):
    s = triples[:, 0].astype(jnp.int32)
    p = triples[:, 1].astype(jnp.int32)
    o = triples[:, 2].astype(jnp.int32)
    so = jnp.concatenate([s, o])

    counts = _sc_counts(so)

    obj_vecs = _prologue(
        objs.astype(jnp.int32)[:, None], attributes.astype(jnp.int32)[:, None],
        angles_gt.astype(jnp.int32)[:, None], boxes_gt,
        params['obj_emb'], params['attr_emb'], params['angle_emb'],
        params['box_w'], params['box_b'][None, :])

    pred_state = None
    for i in range(5):
        w0 = params['g%d_n1w0' % i]
        b02 = params['g%d_n1b0' % i][None, :]
        w1 = params['g%d_n1w1' % i]
        b1 = params['g%d_n1b1' % i]
        usl, ush, uol, uoh = _u_kernel(obj_vecs, w0[0:DIN], w0[2 * DIN:3 * DIN])
        gs, go = _sc_gather(usl, ush, uol, uoh, s, o)
        w0m = w0[DIN:2 * DIN]
        if i == 0:
            nsc, noc, pred_state = _edge_first(
                gs, go, p[:, None], params['pred_emb'], w0m, b02, w1,
                b1[None, :])
        elif i < 4:
            nsc, noc, pred_state = _edge_mid(
                gs, go, pred_state, w0m, b02, w1, b1[None, :])
        else:
            w1so = jnp.concatenate([w1[:, :H], w1[:, H + DIN:]], axis=1)
            b1so2 = jnp.concatenate([b1[:H], b1[H + DIN:]])[None, :]
            nsc, noc = _edge_last(gs, go, pred_state, w0m, b02, w1so, b1so2)
        pooled = _sc_scatter(nsc, noc, s, o)
        obj_vecs = _node(pooled, counts, params['g%d_n2w0' % i],
                         params['g%d_n2b0' % i][None, :],
                         params['g%d_n2w1' % i],
                         params['g%d_n2b1' % i][None, :])

    ws = [params['bmv_w0'], params['bmv_b0'][None, :],
          params['bmv_w1'], params['bmv_b1'][None, :],
          params['bm_w'], params['bm_b'][None, :],
          params['bv_w'], params['bv_b'][None, :],
          params['amv_w0'], params['amv_b0'][None, :],
          params['amv_w1'], params['amv_b1'][None, :],
          params['am_w'], params['am_b'][None, :],
          params['av_w'], params['av_b'][None, :]]
    mu, logvar = _heads(obj_vecs, ws)
    return mu, logvar
